# baseline (speedup 1.0000x reference)
import jax
import jax.numpy as jnp
from jax import lax
from jax.experimental import pallas as pl
from jax.experimental.pallas import tpu as pltpu
from jax.experimental.pallas import tpu_sc as plsc

N = 10000
E = 320000
D = 128

NC = 2
NS = 16
NW = NC * NS

CHUNK = 128
G0 = 160
G1 = 0
GT = NS * (G0 + G1)
NB = 2
NQ = 4
EPAD = GT * CHUNK
N_ACC = 10112
ZROWS = N_ACC // NS


def _sc_body(feat_hbm, src_hbm, dst_hbm, zeros_hbm, out_hbm,
             sslot_v, dslot_v, rows_v, acc_sh, zsem, *sems):
    sem_g = sems[:NB]
    sem_s = sems[NB:2 * NB]
    sem_is = sems[2 * NB:2 * NB + NQ]
    sem_id = sems[2 * NB + NQ:]
    c = lax.axis_index("c")
    s = lax.axis_index("s")

    zcopy = pltpu.async_copy(zeros_hbm.at[pl.ds(s * ZROWS, ZROWS)],
                             acc_sh.at[pl.ds(s * ZROWS, ZROWS)], zsem)

    def pipeline(gc, base):
        def srci_start(i, q):
            pltpu.async_copy(src_hbm.at[base + i], sslot_v.at[q], sem_is[q])

        def srci_wait(i, q):
            pltpu.make_async_copy(src_hbm.at[base + i], sslot_v.at[q],
                                  sem_is[q]).wait()

        def dsti_start(i, q):
            pltpu.async_copy(dst_hbm.at[base + i], dslot_v.at[q], sem_id[q])

        def dsti_wait(i, q):
            pltpu.make_async_copy(dst_hbm.at[base + i], dslot_v.at[q],
                                  sem_id[q]).wait()

        def gather_start(q, b):
            pltpu.async_copy(feat_hbm.at[sslot_v.at[q]], rows_v.at[b],
                             sem_g[b])

        def gather_wait(q, b):
            pltpu.make_async_copy(feat_hbm.at[sslot_v.at[q]], rows_v.at[b],
                                  sem_g[b]).wait()

        def scatter_start(b, q):
            pltpu.async_copy(rows_v.at[b], acc_sh.at[dslot_v.at[q]],
                             sem_s[b], add=True)

        def scatter_wait(b, q):
            pltpu.make_async_copy(rows_v.at[b], acc_sh.at[dslot_v.at[q]],
                                  sem_s[b]).wait()

        def step(i, b, q, q2, reissue2, reissue4):
            gather_wait(q, b)
            if reissue4:
                srci_start(i + NQ, q)
            dsti_wait(i, q)
            scatter_start(b, q)
            scatter_wait(b, q)
            if reissue4:
                dsti_start(i + NQ, q)
            if reissue2:
                srci_wait(i + NB, q2)
                gather_start(q2, b)

        for q in range(NQ):
            srci_start(q, q)
            dsti_start(q, q)
        for b in range(NB):
            srci_wait(b, b)
            gather_start(b, b)
        zcopy.wait()
        plsc.subcore_barrier()

        def outer(k, carry):
            i0 = k * NQ
            for u in range(NQ):
                step(i0 + u, u % NB, u, (u + NB) % NQ, True, True)
            return carry

        lax.fori_loop(0, gc // NQ - 1, outer, 0)

        i0 = gc - NQ
        for u in range(NQ):
            step(i0 + u, u % NB, u, (u + NB) % NQ, u < NB, False)

    pl.when(c == 0)(lambda: pipeline(G0, s * G0))
    if G1 > 0:
        pl.when(c == 1)(lambda: pipeline(G1, NS * G0 + s * G1))
    else:
        def idle_core():
            zcopy.wait()
            plsc.subcore_barrier()
        pl.when(c == 1)(idle_core)

    plsc.subcore_barrier()
    pltpu.sync_copy(acc_sh.at[pl.ds(s * ZROWS, ZROWS)],
                    out_hbm.at[c, pl.ds(s * ZROWS, ZROWS)])


@jax.jit
def _sc_partials(feat, src, dst, zeros):
    mesh = plsc.VectorSubcoreMesh(core_axis_name="c", subcore_axis_name="s")
    return pl.kernel(
        _sc_body,
        out_type=jax.ShapeDtypeStruct((NC, N_ACC, D), jnp.float32),
        mesh=mesh,
        scratch_types=[
            pltpu.VMEM((NQ, CHUNK), jnp.int32),
            pltpu.VMEM((NQ, CHUNK), jnp.int32),
            pltpu.VMEM((NB, CHUNK, D), jnp.float32),
            pltpu.VMEM_SHARED((N_ACC, D), jnp.float32),
            pltpu.SemaphoreType.DMA,
        ] + [pltpu.SemaphoreType.DMA] * (2 * NB + 2 * NQ),
    )(feat, src, dst, zeros)


def _combine_body(p_ref, o_ref):
    o_ref[...] = p_ref[0] + p_ref[1]


@jax.jit
def _combine(partials):
    bn = 1000
    return pl.pallas_call(
        _combine_body,
        grid=(N // bn,),
        in_specs=[pl.BlockSpec((NC, bn, D), lambda i: (0, i, 0))],
        out_specs=pl.BlockSpec((bn, D), lambda i: (i, 0)),
        out_shape=jax.ShapeDtypeStruct((N, D), jnp.float32),
    )(partials)


def kernel(feat, edge_index):
    src = edge_index[0].astype(jnp.int32)
    dst = edge_index[1].astype(jnp.int32)
    pad = EPAD - E
    src = jnp.concatenate([src, jnp.zeros((pad,), jnp.int32)]).reshape(GT, CHUNK)
    dst = jnp.concatenate([dst, jnp.full((pad,), N, jnp.int32)]).reshape(GT, CHUNK)
    zeros = jnp.zeros((N_ACC, D), jnp.float32)
    partials = _sc_partials(feat, src, dst, zeros)
    return _combine(partials)


# speedup vs baseline: 1.2419x; 1.2419x over previous
import jax
import jax.numpy as jnp
from jax import lax
from jax.experimental import pallas as pl
from jax.experimental.pallas import tpu as pltpu
from jax.experimental.pallas import tpu_sc as plsc

N = 10000
E = 320000
D = 128

NC = 2
NS = 16
NW = NC * NS
CHUNK = 128
G = -(-E // (NW * CHUNK))
GT = NW * G
EPAD = GT * CHUNK
N_ACC = 10112
ZROWS = N_ACC // NS


def _sc_body(feat_hbm, src_hbm, dst_hbm, zeros_hbm, out_hbm,
             sidx_v, didx_v, rows_v, acc_sh, zsem, gsem):
    c = lax.axis_index("c")
    s = lax.axis_index("s")

    pltpu.async_copy(zeros_hbm.at[pl.ds(s * ZROWS, ZROWS)],
                     acc_sh.at[pl.ds(s * ZROWS, ZROWS)], zsem).wait()
    plsc.subcore_barrier()

    wid = s * NC + c
    base = wid * G

    @pl.loop(0, G)
    def _chunk(i):
        pltpu.sync_copy(src_hbm.at[base + i], sidx_v)
        pltpu.sync_copy(dst_hbm.at[base + i], didx_v)
        pltpu.async_copy(feat_hbm.at[sidx_v], rows_v, gsem).wait()
        pltpu.sync_copy(rows_v, acc_sh.at[didx_v], add=True)

    plsc.subcore_barrier()
    pltpu.sync_copy(acc_sh.at[pl.ds(s * ZROWS, ZROWS)],
                    out_hbm.at[c, pl.ds(s * ZROWS, ZROWS)])


@jax.jit
def _sc_partials(feat, src, dst, zeros):
    mesh = plsc.VectorSubcoreMesh(core_axis_name="c", subcore_axis_name="s")
    return pl.kernel(
        _sc_body,
        out_type=jax.ShapeDtypeStruct((NC, N_ACC, D), jnp.float32),
        mesh=mesh,
        scratch_types=[
            pltpu.VMEM((CHUNK,), jnp.int32),
            pltpu.VMEM((CHUNK,), jnp.int32),
            pltpu.VMEM((CHUNK, D), jnp.float32),
            pltpu.VMEM_SHARED((N_ACC, D), jnp.float32),
            pltpu.SemaphoreType.DMA,
            pltpu.SemaphoreType.DMA,
        ],
    )(feat, src, dst, zeros)


def _combine_body(p_ref, o_ref):
    o_ref[...] = p_ref[0] + p_ref[1]


@jax.jit
def _combine(partials):
    bn = 1000
    return pl.pallas_call(
        _combine_body,
        grid=(N // bn,),
        in_specs=[pl.BlockSpec((NC, bn, D), lambda i: (0, i, 0))],
        out_specs=pl.BlockSpec((bn, D), lambda i: (i, 0)),
        out_shape=jax.ShapeDtypeStruct((N, D), jnp.float32),
    )(partials)


def kernel(feat, edge_index):
    src = edge_index[0].astype(jnp.int32)
    dst = edge_index[1].astype(jnp.int32)
    pad = EPAD - E
    src = jnp.concatenate([src, jnp.zeros((pad,), jnp.int32)]).reshape(GT, CHUNK)
    dst = jnp.concatenate([dst, jnp.full((pad,), N, jnp.int32)]).reshape(GT, CHUNK)
    zeros = jnp.zeros((N_ACC, D), jnp.float32)
    partials = _sc_partials(feat, src, dst, zeros)
    return _combine(partials)
